# baseline (speedup 1.0000x reference)
import functools
import math

import jax
import jax.numpy as jnp
from jax import lax
from jax.experimental import pallas as pl
from jax.experimental.pallas import tpu as pltpu

try:
    from jax.experimental.pallas import tpu_sc as plsc
except ImportError:
    plsc = None

F32 = jnp.float32


def _mm(x, w, b, act=False, bm=1000):
    M, K = x.shape
    N = w.shape[1]

    def kern(x_ref, w_ref, b_ref, o_ref):
        y = jnp.dot(x_ref[...], w_ref[...], preferred_element_type=F32)
        y = y + b_ref[...]
        if act:
            y = jnp.maximum(y, 0.0)
        o_ref[...] = y

    return pl.pallas_call(
        kern,
        grid=(M // bm,),
        in_specs=[
            pl.BlockSpec((bm, K), lambda i: (i, 0)),
            pl.BlockSpec((K, N), lambda i: (0, 0)),
            pl.BlockSpec((1, N), lambda i: (0, 0)),
        ],
        out_specs=pl.BlockSpec((bm, N), lambda i: (i, 0)),
        out_shape=jax.ShapeDtypeStruct((M, N), F32),
    )(x, w, b.reshape(1, -1))


def _mlp2(x, w1, b1, w2, b2, bm=1000):
    M, K = x.shape
    H = w1.shape[1]
    N = w2.shape[1]

    def kern(x_ref, w1_ref, b1_ref, w2_ref, b2_ref, o_ref):
        h = jnp.dot(x_ref[...], w1_ref[...], preferred_element_type=F32)
        h = jnp.maximum(h + b1_ref[...], 0.0)
        y = jnp.dot(h, w2_ref[...], preferred_element_type=F32)
        o_ref[...] = jnp.maximum(y + b2_ref[...], 0.0)

    return pl.pallas_call(
        kern,
        grid=(M // bm,),
        in_specs=[
            pl.BlockSpec((bm, K), lambda i: (i, 0)),
            pl.BlockSpec((K, H), lambda i: (0, 0)),
            pl.BlockSpec((1, H), lambda i: (0, 0)),
            pl.BlockSpec((H, N), lambda i: (0, 0)),
            pl.BlockSpec((1, N), lambda i: (0, 0)),
        ],
        out_specs=pl.BlockSpec((bm, N), lambda i: (i, 0)),
        out_shape=jax.ShapeDtypeStruct((M, N), F32),
    )(x, w1, b1.reshape(1, -1), w2, b2.reshape(1, -1))


_BE = 2000


def _combine(qsrc, q_ci, kvsrc, kv_ci, elf, esm, a64, a5, scale):
    E = elf.shape[0]
    C = 128

    def kern(q_ref, kv_ref, f_ref, s_ref, a64_ref, a5_ref, o_ref, x_ref):
        e = jnp.dot(f_ref[...], a64_ref[...], preferred_element_type=F32)
        e = e + jnp.dot(s_ref[...], a5_ref[...], preferred_element_type=F32)
        ke = kv_ref[:, :C]
        ve = kv_ref[:, C:]
        a = jnp.sum(q_ref[...] * (ke + e), axis=1) * scale
        ex = jnp.exp(a)
        o_ref[...] = (ve + e) * ex[:, None]
        li = lax.broadcasted_iota(jnp.int32, (_BE, 128), 1)
        x_ref[...] = jnp.where(li == 0, ex[:, None], 0.0) + jnp.where(
            li == 1, 1.0, 0.0)

    return pl.pallas_call(
        kern,
        grid=(E // _BE,),
        in_specs=[
            pl.BlockSpec((_BE, 128), lambda i, c=q_ci: (i, c)),
            pl.BlockSpec((_BE, 256), lambda i, c=kv_ci: (i, c)),
            pl.BlockSpec((_BE, 64), lambda i: (i, 0)),
            pl.BlockSpec((_BE, 5), lambda i: (i, 0)),
            pl.BlockSpec((64, 128), lambda i: (0, 0)),
            pl.BlockSpec((5, 128), lambda i: (0, 0)),
        ],
        out_specs=[
            pl.BlockSpec((_BE, 128), lambda i: (i, 0)),
            pl.BlockSpec((_BE, 128), lambda i: (i, 0)),
        ],
        out_shape=[
            jax.ShapeDtypeStruct((E, 128), F32),
            jax.ShapeDtypeStruct((E, 128), F32),
        ],
    )(qsrc, kvsrc, elf, esm, a64, a5)


def _node_epilogue(parts, xd, wskip, bskip, lnw, lnb, bm=1000):
    N = xd.shape[0]
    K = xd.shape[1]
    T = N // bm

    def kern_a(u0_ref, u1_ref, t0_ref, t1_ref, xd_ref, w_ref, b_ref,
               y_ref, pt_ref):
        U = u0_ref[0, 0] + u1_ref[0, 0]
        tail = t0_ref[0, 0] + t1_ref[0, 0]
        S = tail[:, 0:1]
        deg = tail[:, 1:2]
        agg = U / (S + 1e-16) / jnp.maximum(deg, 1.0)
        y = agg + jnp.dot(xd_ref[...], w_ref[...], preferred_element_type=F32) + b_ref[...]
        y_ref[...] = y
        sy = jnp.sum(y)
        sq = jnp.sum(y * y)
        li = lax.broadcasted_iota(jnp.int32, (1, 1, 128), 2)
        pt_ref[...] = jnp.where(li == 0, sy, 0.0) + jnp.where(li == 1, sq, 0.0)

    y, part = pl.pallas_call(
        kern_a,
        grid=(T,),
        in_specs=[
            pl.BlockSpec((1, 1, bm, 128), lambda i: (0, 0, i, 0)),
            pl.BlockSpec((1, 1, bm, 128), lambda i: (1, 0, i, 0)),
            pl.BlockSpec((1, 1, bm, 128), lambda i: (0, 1, i, 0)),
            pl.BlockSpec((1, 1, bm, 128), lambda i: (1, 1, i, 0)),
            pl.BlockSpec((bm, K), lambda i: (i, 0)),
            pl.BlockSpec((K, 128), lambda i: (0, 0)),
            pl.BlockSpec((1, 128), lambda i: (0, 0)),
        ],
        out_specs=[
            pl.BlockSpec((bm, 128), lambda i: (i, 0)),
            pl.BlockSpec((1, 1, 128), lambda i: (i, 0, 0)),
        ],
        out_shape=[
            jax.ShapeDtypeStruct((N, 128), F32),
            jax.ShapeDtypeStruct((T, 1, 128), F32),
        ],
    )(parts, parts, parts, parts, xd, wskip, bskip.reshape(1, -1))
    return _ln_apply(y, part, lnw, lnb, bm)


def _ln_apply(y, part, lnw, lnb, bm):
    N, D = y.shape
    T = part.shape[0]
    cnt = float(N * D)

    def kern(y_ref, pt_ref, w_ref, b_ref, o_ref):
        sy = jnp.sum(pt_ref[:, :, 0:1])
        sq = jnp.sum(pt_ref[:, :, 1:2])
        m = sy / cnt
        var = jnp.maximum(sq / cnt - m * m, 0.0)
        inv = 1.0 / (jnp.sqrt(var) + 1e-5)
        o_ref[...] = jnp.maximum((y_ref[...] - m) * inv * w_ref[..., :D] + b_ref[..., :D], 0.0)

    return pl.pallas_call(
        kern,
        grid=(N // bm,),
        in_specs=[
            pl.BlockSpec((bm, D), lambda i: (i, 0)),
            pl.BlockSpec((T, 1, 128), lambda i: (0, 0, 0)),
            pl.BlockSpec((1, 128), lambda i: (0, 0)),
            pl.BlockSpec((1, 128), lambda i: (0, 0)),
        ],
        out_specs=pl.BlockSpec((bm, D), lambda i: (i, 0)),
        out_shape=jax.ShapeDtypeStruct((N, D), F32),
    )(y, part, _pad128(lnw).reshape(1, -1), _pad128(lnb).reshape(1, -1))


def _pad128(v):
    d = v.shape[0]
    if d >= 128:
        return v
    return jnp.pad(v, (0, 128 - d))


def _edge_mlp(elf, esm, hvsrc, hv_ci, hv_w, hv_off, hcsrc, hc_ci, hc_w,
              hc_off, wf, w5, wv, wc, b1, w2, b2, want_stats):
    E = elf.shape[0]
    OE = w2.shape[1]
    T = E // _BE

    def kern(f_ref, s_ref, hv_ref, hc_ref, wf_ref, w5_ref, wv_ref, wc_ref,
             b1_ref, w2_ref, b2_ref, y_ref, pt_ref):
        h = jnp.dot(f_ref[...], wf_ref[...], preferred_element_type=F32)
        h = h + jnp.dot(s_ref[...], w5_ref[...], preferred_element_type=F32)
        h = h + jnp.dot(hv_ref[:, hv_off:hv_off + 64], wv_ref[...],
                        preferred_element_type=F32)
        h = h + jnp.dot(hc_ref[:, hc_off:hc_off + 64], wc_ref[...],
                        preferred_element_type=F32)
        h = jnp.maximum(h + b1_ref[...], 0.0)
        y = jnp.dot(h, w2_ref[...], preferred_element_type=F32) + b2_ref[...]
        y_ref[...] = y
        if want_stats:
            sy = jnp.sum(y)
            sq = jnp.sum(y * y)
            li = lax.broadcasted_iota(jnp.int32, (1, 1, 128), 2)
            pt_ref[...] = jnp.where(li == 0, sy, 0.0) + jnp.where(li == 1, sq, 0.0)
        else:
            pt_ref[...] = jnp.zeros((1, 1, 128), F32)

    return pl.pallas_call(
        kern,
        grid=(T,),
        in_specs=[
            pl.BlockSpec((_BE, 64), lambda i: (i, 0)),
            pl.BlockSpec((_BE, 5), lambda i: (i, 0)),
            pl.BlockSpec((_BE, hv_w), lambda i, c=hv_ci: (i, c)),
            pl.BlockSpec((_BE, hc_w), lambda i, c=hc_ci: (i, c)),
            pl.BlockSpec((64, 64), lambda i: (0, 0)),
            pl.BlockSpec((5, 64), lambda i: (0, 0)),
            pl.BlockSpec((64, 64), lambda i: (0, 0)),
            pl.BlockSpec((64, 64), lambda i: (0, 0)),
            pl.BlockSpec((1, 64), lambda i: (0, 0)),
            pl.BlockSpec((64, OE), lambda i: (0, 0)),
            pl.BlockSpec((1, OE), lambda i: (0, 0)),
        ],
        out_specs=[
            pl.BlockSpec((_BE, OE), lambda i: (i, 0)),
            pl.BlockSpec((1, 1, 128), lambda i: (i, 0, 0)),
        ],
        out_shape=[
            jax.ShapeDtypeStruct((E, OE), F32),
            jax.ShapeDtypeStruct((T, 1, 128), F32),
        ],
    )(elf, esm, hvsrc, hcsrc, wf, w5, wv, wc, b1.reshape(1, -1), w2,
      b2.reshape(1, -1))


_SC_CORES = 2
_SC_SUBCORES = 16
_SC_W = _SC_CORES * _SC_SUBCORES
_CH = 80


def _gather_rows(table, idx2d):
    N, D = table.shape
    W, steps, _ = idx2d.shape
    E = W * steps * _CH
    per_w = E // _SC_W
    mesh = plsc.VectorSubcoreMesh(core_axis_name="c", subcore_axis_name="s")

    @functools.partial(
        pl.kernel,
        mesh=mesh,
        out_type=jax.ShapeDtypeStruct((E, D), F32),
        scratch_types=[
            pltpu.VMEM((steps, _CH), jnp.int32),
            pltpu.VMEM((_CH, D), F32),
            pltpu.VMEM((_CH, D), F32),
            pltpu.SemaphoreType.DMA,
            pltpu.SemaphoreType.DMA,
        ],
    )
    def k(table_hbm, idx_hbm, out_hbm, idx_v, rows0, rows1, sem0, sem1):
        wid = lax.axis_index("s") * _SC_CORES + lax.axis_index("c")
        base = wid * per_w
        pltpu.sync_copy(idx_hbm.at[wid], idx_v)
        bufs = (rows0, rows1)
        sems = (sem0, sem1)

        def fire(j, b):
            pltpu.async_copy(table_hbm.at[idx_v.at[j]], bufs[b], sems[b])

        fire(0, 0)
        fire(1, 1)

        def body(g, carry):
            for b in range(2):
                j = 2 * g + b
                pltpu.make_async_copy(
                    table_hbm.at[pl.ds(0, _CH)], bufs[b], sems[b]).wait()
                pltpu.sync_copy(bufs[b], out_hbm.at[pl.ds(base + j * _CH, _CH)])

                @pl.when(j + 2 < steps)
                def _():
                    fire(j + 2, b)
            return carry

        lax.fori_loop(0, steps // 2, body, 0)
        if steps % 2:
            j = steps - 1
            pltpu.make_async_copy(
                table_hbm.at[pl.ds(0, _CH)], bufs[j % 2], sems[j % 2]).wait()
            pltpu.sync_copy(bufs[j % 2], out_hbm.at[pl.ds(base + j * _CH, _CH)])

    return k(table, idx2d)


def _scatter_partials(rows, ex2d, idx2d, N):
    E, D = rows.shape
    per_w = E // _SC_W
    steps = idx2d.shape[1]
    cp = (N // _SC_SUBCORES) & ~7
    rem = N - cp * _SC_SUBCORES
    zeros = jnp.zeros((N, 128), F32)
    mesh = plsc.VectorSubcoreMesh(core_axis_name="c", subcore_axis_name="s")

    @functools.partial(
        pl.kernel,
        mesh=mesh,
        out_type=jax.ShapeDtypeStruct((_SC_CORES, 2, N, 128), F32),
        scratch_types=[
            pltpu.VMEM((steps, _CH), jnp.int32),
            pltpu.VMEM((_CH, 128), F32),
            pltpu.VMEM((_CH, 128), F32),
            pltpu.VMEM_SHARED((N, 128), F32),
            pltpu.SemaphoreType.DMA,
            pltpu.SemaphoreType.DMA,
            pltpu.SemaphoreType.DMA,
            pltpu.SemaphoreType.DMA,
        ],
    )
    def k(rows_hbm, ex_hbm, idx_hbm, zero_hbm, out_hbm,
          idx_v, rows0, rows1, acc_sh, sem0, sem1, asem0, asem1):
        cid = lax.axis_index("c")
        sid = lax.axis_index("s")
        wid = sid * _SC_CORES + cid
        base = wid * per_w
        pltpu.sync_copy(idx_hbm.at[wid], idx_v)
        bufs = (rows0, rows1)
        sems = (sem0, sem1)
        asems = (asem0, asem1)

        def copy_out(p):
            dst = out_hbm.at[cid].at[p]
            pltpu.sync_copy(
                acc_sh.at[pl.ds(sid * cp, cp)],
                dst.at[pl.ds(sid * cp, cp)],
            )
            if rem:
                @pl.when(sid == 0)
                def _():
                    pltpu.sync_copy(
                        acc_sh.at[pl.ds(cp * _SC_SUBCORES, rem)],
                        dst.at[pl.ds(cp * _SC_SUBCORES, rem)],
                    )

        for p, src in enumerate((rows_hbm, ex_hbm)):
            @pl.when(sid == 0)
            def _():
                pltpu.sync_copy(zero_hbm, acc_sh)

            plsc.subcore_barrier()

            def fire(j, b, src=src):
                pltpu.async_copy(
                    src.at[pl.ds(base + j * _CH, _CH)], bufs[b], sems[b])

            def drain_add(j, b, src=src):
                pltpu.make_async_copy(
                    src.at[pl.ds(0, _CH)], bufs[b], sems[b]).wait()
                pltpu.async_copy(
                    bufs[b], acc_sh.at[idx_v.at[j]], asems[b], add=True)

            def wait_add(b, src=src):
                pltpu.make_async_copy(
                    src.at[pl.ds(0, _CH)], bufs[b], asems[b]).wait()

            fire(0, 0)
            fire(1, 1)

            def body(g, carry, fire=fire, drain_add=drain_add,
                     wait_add=wait_add):
                for b in range(2):
                    j = 2 * g + b
                    drain_add(j, b)

                    @pl.when(j + 2 < steps)
                    def _():
                        wait_add(b)
                        fire(j + 2, b)
                return carry

            lax.fori_loop(0, steps // 2, body, 0)
            if steps % 2:
                drain_add(steps - 1, (steps - 1) % 2)
                wait_add((steps - 1) % 2)
                wait_add((steps - 2) % 2)
            else:
                wait_add(0)
                wait_add(1)
            plsc.subcore_barrier()
            copy_out(p)
            plsc.subcore_barrier()

    return k(rows, ex2d, idx2d, zeros)


def _tconv_pass(q_tbl_src, q_ci, kv_src, kv_ci, elf, esm, a64, a5, didx3, nd):
    scale = 1.0 / math.sqrt(128.0)
    U, ex = _combine(q_tbl_src, q_ci, kv_src, kv_ci, elf, esm, a64, a5, scale)
    return _scatter_partials(U, ex, didx3, nd)


def _pad_out(w, b):
    return jnp.pad(w, ((0, 0), (0, 64))), jnp.pad(b, (0, 64))


def kernel(solvers, var_lp_f, con_lp_f, lo_costs, hi_costs, def_mm,
           edge_lp_f_wo_ss, var_learned_f, con_learned_f, edge_learned_f,
           edge_index_var_con, params, num_dual_iterations):
    vi = edge_index_var_con[0].reshape(_SC_W, -1, _CH)
    ci = edge_index_var_con[1].reshape(_SC_W, -1, _CH)

    vlf = var_learned_f
    clf = con_learned_f
    elf = edge_learned_f
    esm = jnp.concatenate(
        [lo_costs[:, None], hi_costs[:, None], edge_lp_f_wo_ss], axis=1)

    pred = params["pred"]
    layers = params["layers"]
    n_layers = len(layers)

    def _kvq_tables(lp, vc_in, cc_in):
        con, var = lp["con"], lp["var"]
        w_kvq_v = jnp.concatenate(
            [con["k"]["w"], con["v"]["w"], var["q"]["w"]], axis=1)
        b_kvq_v = jnp.concatenate(
            [con["k"]["b"], con["v"]["b"], var["q"]["b"]])
        return (_mm(vc_in, w_kvq_v, b_kvq_v),
                _mm(cc_in, lp["con"]["q"]["w"], lp["con"]["q"]["b"]))

    vc = jnp.concatenate([vlf, var_lp_f], axis=1)
    cc = jnp.concatenate([clf, con_lp_f], axis=1)
    tbl_v, tbl_qc = _kvq_tables(layers[0], vc, cc)
    g_v_src = _gather_rows(tbl_v, vi)
    kv_col, qv_col = 0, 2
    g_qc_src = _gather_rows(tbl_qc, ci)
    qc_col = 0

    for li, lp in enumerate(layers):
        last = li == n_layers - 1
        nxt = None if last else layers[li + 1]
        vc = jnp.concatenate([vlf, var_lp_f], axis=1)
        cc = jnp.concatenate([clf, con_lp_f], axis=1)
        con, var, edge = lp["con"], lp["var"], lp["edge"]

        we = con["e"]["w"]
        pc = _tconv_pass(g_qc_src, qc_col, g_v_src, kv_col, elf, esm,
                         we[:64], we[64:69], ci, cc.shape[0])
        clf = _node_epilogue(pc, cc, con["skip"]["w"],
                             con["skip"]["b"], lp["cn"]["w"], lp["cn"]["b"])
        cc = jnp.concatenate([clf, con_lp_f], axis=1)

        w_kv_c = jnp.concatenate([var["k"]["w"], var["v"]["w"]], axis=1)
        b_kv_c = jnp.concatenate([var["k"]["b"], var["v"]["b"]])
        tbl_kvc = _mm(cc, w_kv_c, b_kv_c)
        if last:
            hc = _mlp2(cc, edge["c1"]["w"], edge["c1"]["b"],
                       edge["c2"]["w"], edge["c2"]["b"])
            hcp = _mlp2(cc, pred["c1"]["w"], pred["c1"]["b"],
                        pred["c2"]["w"], pred["c2"]["b"])
            tbl_c = jnp.concatenate([tbl_kvc, hc, hcp], axis=1)
        else:
            c2w, c2b = _pad_out(edge["c2"]["w"], edge["c2"]["b"])
            hc = _mlp2(cc, edge["c1"]["w"], edge["c1"]["b"], c2w, c2b)
            tqn = _mm(cc, nxt["con"]["q"]["w"], nxt["con"]["q"]["b"])
            tbl_c = jnp.concatenate([tbl_kvc, hc, tqn], axis=1)
        g_c = _gather_rows(tbl_c, ci)

        we = var["e"]["w"]
        pv = _tconv_pass(g_v_src, qv_col, g_c, 0, elf, esm, we[:64],
                         we[64:69], vi, vc.shape[0])
        vlf = _node_epilogue(pv, vc, var["skip"]["w"],
                             var["skip"]["b"], lp["vn"]["w"], lp["vn"]["b"])
        vc = jnp.concatenate([vlf, var_lp_f], axis=1)

        if last:
            hva = _mlp2(vc, edge["v1"]["w"], edge["v1"]["b"],
                        edge["v2"]["w"], edge["v2"]["b"])
            hvp = _mlp2(vc, pred["v1"]["w"], pred["v1"]["b"],
                        pred["v2"]["w"], pred["v2"]["b"])
            tbl_hv = jnp.concatenate([hva, hvp], axis=1)
            hv_ci = 0
        else:
            v2w, v2b = _pad_out(edge["v2"]["w"], edge["v2"]["b"])
            hv128 = _mlp2(vc, edge["v1"]["w"], edge["v1"]["b"], v2w, v2b)
            tkvqn, tqcn_unused = None, None
            tbl_vn, _ = _kvq_tables(nxt, vc, cc)
            tbl_hv = jnp.concatenate([tbl_vn, hv128], axis=1)
            hv_ci = 3
        g_hv = _gather_rows(tbl_hv, vi)

        e1 = edge["e1"]["w"]
        y, part = _edge_mlp(elf, esm, g_hv, hv_ci, 128, 0, g_c, 2, 128, 0,
                            e1[:64], e1[64:69], e1[69:133], e1[133:197],
                            edge["e1"]["b"], edge["e2"]["w"], edge["e2"]["b"],
                            want_stats=True)
        elf = _ln_apply(y, part, lp["en"]["w"], lp["en"]["b"], _BE)

        if not last:
            g_v_src, kv_col, qv_col = g_hv, 0, 2
            g_qc_src, qc_col = g_c, 3

    e1 = pred["e1"]["w"]
    y, _ = _edge_mlp(elf, esm, g_hv, 0, 128, 64, g_c, 2, 128, 64,
                     e1[:64], e1[64:69], e1[69:133], e1[133:197],
                     pred["e1"]["b"], pred["e2"]["w"], pred["e2"]["b"],
                     want_stats=False)
    return y


# speedup vs baseline: 1.0668x; 1.0668x over previous
import functools
import math

import jax
import jax.numpy as jnp
from jax import lax
from jax.experimental import pallas as pl
from jax.experimental.pallas import tpu as pltpu

try:
    from jax.experimental.pallas import tpu_sc as plsc
except ImportError:
    plsc = None

F32 = jnp.float32


def _mm(x, w, b, act=False, bm=1000):
    M, K = x.shape
    N = w.shape[1]

    def kern(x_ref, w_ref, b_ref, o_ref):
        y = jnp.dot(x_ref[...], w_ref[...], preferred_element_type=F32)
        y = y + b_ref[...]
        if act:
            y = jnp.maximum(y, 0.0)
        o_ref[...] = y

    return pl.pallas_call(
        kern,
        grid=(M // bm,),
        in_specs=[
            pl.BlockSpec((bm, K), lambda i: (i, 0)),
            pl.BlockSpec((K, N), lambda i: (0, 0)),
            pl.BlockSpec((1, N), lambda i: (0, 0)),
        ],
        out_specs=pl.BlockSpec((bm, N), lambda i: (i, 0)),
        out_shape=jax.ShapeDtypeStruct((M, N), F32),
    )(x, w, b.reshape(1, -1))


def _mlp2(x, w1, b1, w2, b2, bm=1000):
    M, K = x.shape
    H = w1.shape[1]
    N = w2.shape[1]

    def kern(x_ref, w1_ref, b1_ref, w2_ref, b2_ref, o_ref):
        h = jnp.dot(x_ref[...], w1_ref[...], preferred_element_type=F32)
        h = jnp.maximum(h + b1_ref[...], 0.0)
        y = jnp.dot(h, w2_ref[...], preferred_element_type=F32)
        o_ref[...] = jnp.maximum(y + b2_ref[...], 0.0)

    return pl.pallas_call(
        kern,
        grid=(M // bm,),
        in_specs=[
            pl.BlockSpec((bm, K), lambda i: (i, 0)),
            pl.BlockSpec((K, H), lambda i: (0, 0)),
            pl.BlockSpec((1, H), lambda i: (0, 0)),
            pl.BlockSpec((H, N), lambda i: (0, 0)),
            pl.BlockSpec((1, N), lambda i: (0, 0)),
        ],
        out_specs=pl.BlockSpec((bm, N), lambda i: (i, 0)),
        out_shape=jax.ShapeDtypeStruct((M, N), F32),
    )(x, w1, b1.reshape(1, -1), w2, b2.reshape(1, -1))


_BE = 4000


def _combine(qsrc, q_ci, kvsrc, kv_ci, elf, esm, a64, a5, scale):
    E = elf.shape[0]
    C = 128

    def kern(q_ref, kv_ref, f_ref, s_ref, a64_ref, a5_ref, o_ref, x_ref):
        e = jnp.dot(f_ref[...], a64_ref[...], preferred_element_type=F32)
        e = e + jnp.dot(s_ref[...], a5_ref[...], preferred_element_type=F32)
        ke = kv_ref[:, :C]
        ve = kv_ref[:, C:]
        a = jnp.sum(q_ref[...] * (ke + e), axis=1) * scale
        ex = jnp.exp(a)
        o_ref[...] = (ve + e) * ex[:, None]
        li = lax.broadcasted_iota(jnp.int32, (_BE, 128), 1)
        x_ref[...] = jnp.where(li == 0, ex[:, None], 0.0) + jnp.where(
            li == 1, 1.0, 0.0)

    return pl.pallas_call(
        kern,
        grid=(E // _BE,),
        in_specs=[
            pl.BlockSpec((_BE, 128), lambda i, c=q_ci: (i, c)),
            pl.BlockSpec((_BE, 256), lambda i, c=kv_ci: (i, c)),
            pl.BlockSpec((_BE, 64), lambda i: (i, 0)),
            pl.BlockSpec((_BE, 5), lambda i: (i, 0)),
            pl.BlockSpec((64, 128), lambda i: (0, 0)),
            pl.BlockSpec((5, 128), lambda i: (0, 0)),
        ],
        out_specs=[
            pl.BlockSpec((_BE, 128), lambda i: (i, 0)),
            pl.BlockSpec((_BE, 128), lambda i: (i, 0)),
        ],
        out_shape=[
            jax.ShapeDtypeStruct((E, 128), F32),
            jax.ShapeDtypeStruct((E, 128), F32),
        ],
    )(qsrc, kvsrc, elf, esm, a64, a5)


def _node_epilogue(parts, xd, wskip, bskip, lnw, lnb, bm=1000):
    N = xd.shape[0]
    K = xd.shape[1]
    T = N // bm

    def kern_a(u0_ref, u1_ref, t0_ref, t1_ref, xd_ref, w_ref, b_ref,
               y_ref, pt_ref):
        U = u0_ref[0, 0] + u1_ref[0, 0]
        tail = t0_ref[0, 0] + t1_ref[0, 0]
        S = tail[:, 0:1]
        deg = tail[:, 1:2]
        agg = U / (S + 1e-16) / jnp.maximum(deg, 1.0)
        y = agg + jnp.dot(xd_ref[...], w_ref[...], preferred_element_type=F32) + b_ref[...]
        y_ref[...] = y
        sy = jnp.sum(y)
        sq = jnp.sum(y * y)
        li = lax.broadcasted_iota(jnp.int32, (1, 1, 128), 2)
        pt_ref[...] = jnp.where(li == 0, sy, 0.0) + jnp.where(li == 1, sq, 0.0)

    y, part = pl.pallas_call(
        kern_a,
        grid=(T,),
        in_specs=[
            pl.BlockSpec((1, 1, bm, 128), lambda i: (0, 0, i, 0)),
            pl.BlockSpec((1, 1, bm, 128), lambda i: (1, 0, i, 0)),
            pl.BlockSpec((1, 1, bm, 128), lambda i: (0, 1, i, 0)),
            pl.BlockSpec((1, 1, bm, 128), lambda i: (1, 1, i, 0)),
            pl.BlockSpec((bm, K), lambda i: (i, 0)),
            pl.BlockSpec((K, 128), lambda i: (0, 0)),
            pl.BlockSpec((1, 128), lambda i: (0, 0)),
        ],
        out_specs=[
            pl.BlockSpec((bm, 128), lambda i: (i, 0)),
            pl.BlockSpec((1, 1, 128), lambda i: (i, 0, 0)),
        ],
        out_shape=[
            jax.ShapeDtypeStruct((N, 128), F32),
            jax.ShapeDtypeStruct((T, 1, 128), F32),
        ],
    )(parts, parts, parts, parts, xd, wskip, bskip.reshape(1, -1))
    return _ln_apply(y, part, lnw, lnb, bm)


def _ln_apply(y, part, lnw, lnb, bm):
    N, D = y.shape
    T = part.shape[0]
    cnt = float(N * D)

    def kern(y_ref, pt_ref, w_ref, b_ref, o_ref):
        sy = jnp.sum(pt_ref[:, :, 0:1])
        sq = jnp.sum(pt_ref[:, :, 1:2])
        m = sy / cnt
        var = jnp.maximum(sq / cnt - m * m, 0.0)
        inv = 1.0 / (jnp.sqrt(var) + 1e-5)
        o_ref[...] = jnp.maximum((y_ref[...] - m) * inv * w_ref[..., :D] + b_ref[..., :D], 0.0)

    return pl.pallas_call(
        kern,
        grid=(N // bm,),
        in_specs=[
            pl.BlockSpec((bm, D), lambda i: (i, 0)),
            pl.BlockSpec((T, 1, 128), lambda i: (0, 0, 0)),
            pl.BlockSpec((1, 128), lambda i: (0, 0)),
            pl.BlockSpec((1, 128), lambda i: (0, 0)),
        ],
        out_specs=pl.BlockSpec((bm, D), lambda i: (i, 0)),
        out_shape=jax.ShapeDtypeStruct((N, D), F32),
    )(y, part, _pad128(lnw).reshape(1, -1), _pad128(lnb).reshape(1, -1))


def _pad128(v):
    d = v.shape[0]
    if d >= 128:
        return v
    return jnp.pad(v, (0, 128 - d))


def _edge_mlp(elf, esm, hvsrc, hv_ci, hv_w, hv_off, hcsrc, hc_ci, hc_w,
              hc_off, wf, w5, wv, wc, b1, w2, b2, want_stats):
    E = elf.shape[0]
    OE = w2.shape[1]
    T = E // _BE

    def kern(f_ref, s_ref, hv_ref, hc_ref, wf_ref, w5_ref, wv_ref, wc_ref,
             b1_ref, w2_ref, b2_ref, y_ref, pt_ref):
        h = jnp.dot(f_ref[...], wf_ref[...], preferred_element_type=F32)
        h = h + jnp.dot(s_ref[...], w5_ref[...], preferred_element_type=F32)
        h = h + jnp.dot(hv_ref[:, hv_off:hv_off + 64], wv_ref[...],
                        preferred_element_type=F32)
        h = h + jnp.dot(hc_ref[:, hc_off:hc_off + 64], wc_ref[...],
                        preferred_element_type=F32)
        h = jnp.maximum(h + b1_ref[...], 0.0)
        y = jnp.dot(h, w2_ref[...], preferred_element_type=F32) + b2_ref[...]
        y_ref[...] = y
        if want_stats:
            sy = jnp.sum(y)
            sq = jnp.sum(y * y)
            li = lax.broadcasted_iota(jnp.int32, (1, 1, 128), 2)
            pt_ref[...] = jnp.where(li == 0, sy, 0.0) + jnp.where(li == 1, sq, 0.0)
        else:
            pt_ref[...] = jnp.zeros((1, 1, 128), F32)

    return pl.pallas_call(
        kern,
        grid=(T,),
        in_specs=[
            pl.BlockSpec((_BE, 64), lambda i: (i, 0)),
            pl.BlockSpec((_BE, 5), lambda i: (i, 0)),
            pl.BlockSpec((_BE, hv_w), lambda i, c=hv_ci: (i, c)),
            pl.BlockSpec((_BE, hc_w), lambda i, c=hc_ci: (i, c)),
            pl.BlockSpec((64, 64), lambda i: (0, 0)),
            pl.BlockSpec((5, 64), lambda i: (0, 0)),
            pl.BlockSpec((64, 64), lambda i: (0, 0)),
            pl.BlockSpec((64, 64), lambda i: (0, 0)),
            pl.BlockSpec((1, 64), lambda i: (0, 0)),
            pl.BlockSpec((64, OE), lambda i: (0, 0)),
            pl.BlockSpec((1, OE), lambda i: (0, 0)),
        ],
        out_specs=[
            pl.BlockSpec((_BE, OE), lambda i: (i, 0)),
            pl.BlockSpec((1, 1, 128), lambda i: (i, 0, 0)),
        ],
        out_shape=[
            jax.ShapeDtypeStruct((E, OE), F32),
            jax.ShapeDtypeStruct((T, 1, 128), F32),
        ],
    )(elf, esm, hvsrc, hcsrc, wf, w5, wv, wc, b1.reshape(1, -1), w2,
      b2.reshape(1, -1))


_SC_CORES = 2
_SC_SUBCORES = 16
_SC_W = _SC_CORES * _SC_SUBCORES
_CH = 80


def _gather_rows(table, idx2d):
    N, D = table.shape
    W, steps, _ = idx2d.shape
    E = W * steps * _CH
    per_w = E // _SC_W
    mesh = plsc.VectorSubcoreMesh(core_axis_name="c", subcore_axis_name="s")

    @functools.partial(
        pl.kernel,
        mesh=mesh,
        out_type=jax.ShapeDtypeStruct((E, D), F32),
        scratch_types=[
            pltpu.VMEM((steps, _CH), jnp.int32),
            pltpu.VMEM((_CH, D), F32),
            pltpu.VMEM((_CH, D), F32),
            pltpu.SemaphoreType.DMA,
            pltpu.SemaphoreType.DMA,
        ],
    )
    def k(table_hbm, idx_hbm, out_hbm, idx_v, rows0, rows1, sem0, sem1):
        wid = lax.axis_index("s") * _SC_CORES + lax.axis_index("c")
        base = wid * per_w
        pltpu.sync_copy(idx_hbm.at[wid], idx_v)
        bufs = (rows0, rows1)
        sems = (sem0, sem1)

        def fire(j, b):
            pltpu.async_copy(table_hbm.at[idx_v.at[j]], bufs[b], sems[b])

        fire(0, 0)
        fire(1, 1)

        def body(g, carry):
            for b in range(2):
                j = 2 * g + b
                pltpu.make_async_copy(
                    table_hbm.at[pl.ds(0, _CH)], bufs[b], sems[b]).wait()
                pltpu.sync_copy(bufs[b], out_hbm.at[pl.ds(base + j * _CH, _CH)])

                @pl.when(j + 2 < steps)
                def _():
                    fire(j + 2, b)
            return carry

        lax.fori_loop(0, steps // 2, body, 0)
        if steps % 2:
            j = steps - 1
            pltpu.make_async_copy(
                table_hbm.at[pl.ds(0, _CH)], bufs[j % 2], sems[j % 2]).wait()
            pltpu.sync_copy(bufs[j % 2], out_hbm.at[pl.ds(base + j * _CH, _CH)])

    return k(table, idx2d)


def _scatter_partials(rows, ex2d, idx2d, N):
    E, D = rows.shape
    per_w = E // _SC_W
    steps = idx2d.shape[1]
    cp = (N // _SC_SUBCORES) & ~7
    rem = N - cp * _SC_SUBCORES
    zeros = jnp.zeros((N, 128), F32)
    mesh = plsc.VectorSubcoreMesh(core_axis_name="c", subcore_axis_name="s")

    @functools.partial(
        pl.kernel,
        mesh=mesh,
        out_type=jax.ShapeDtypeStruct((_SC_CORES, 2, N, 128), F32),
        scratch_types=[
            pltpu.VMEM((steps, _CH), jnp.int32),
            pltpu.VMEM((_CH, 128), F32),
            pltpu.VMEM((_CH, 128), F32),
            pltpu.VMEM_SHARED((N, 128), F32),
            pltpu.SemaphoreType.DMA,
            pltpu.SemaphoreType.DMA,
            pltpu.SemaphoreType.DMA,
            pltpu.SemaphoreType.DMA,
        ],
    )
    def k(rows_hbm, ex_hbm, idx_hbm, zero_hbm, out_hbm,
          idx_v, rows0, rows1, acc_sh, sem0, sem1, asem0, asem1):
        cid = lax.axis_index("c")
        sid = lax.axis_index("s")
        wid = sid * _SC_CORES + cid
        base = wid * per_w
        pltpu.sync_copy(idx_hbm.at[wid], idx_v)
        bufs = (rows0, rows1)
        sems = (sem0, sem1)
        asems = (asem0, asem1)

        def copy_out(p):
            dst = out_hbm.at[cid].at[p]
            pltpu.sync_copy(
                acc_sh.at[pl.ds(sid * cp, cp)],
                dst.at[pl.ds(sid * cp, cp)],
            )
            if rem:
                @pl.when(sid == 0)
                def _():
                    pltpu.sync_copy(
                        acc_sh.at[pl.ds(cp * _SC_SUBCORES, rem)],
                        dst.at[pl.ds(cp * _SC_SUBCORES, rem)],
                    )

        for p, src in enumerate((rows_hbm, ex_hbm)):
            @pl.when(sid == 0)
            def _():
                pltpu.sync_copy(zero_hbm, acc_sh)

            plsc.subcore_barrier()

            def fire(j, b, src=src):
                pltpu.async_copy(
                    src.at[pl.ds(base + j * _CH, _CH)], bufs[b], sems[b])

            def drain_add(j, b, src=src):
                pltpu.make_async_copy(
                    src.at[pl.ds(0, _CH)], bufs[b], sems[b]).wait()
                pltpu.async_copy(
                    bufs[b], acc_sh.at[idx_v.at[j]], asems[b], add=True)

            def wait_add(b, src=src):
                pltpu.make_async_copy(
                    src.at[pl.ds(0, _CH)], bufs[b], asems[b]).wait()

            fire(0, 0)
            fire(1, 1)

            def body(g, carry, fire=fire, drain_add=drain_add,
                     wait_add=wait_add):
                for b in range(2):
                    j = 2 * g + b
                    drain_add(j, b)

                    @pl.when(j + 2 < steps)
                    def _():
                        wait_add(b)
                        fire(j + 2, b)
                return carry

            lax.fori_loop(0, steps // 2, body, 0)
            if steps % 2:
                drain_add(steps - 1, (steps - 1) % 2)
                wait_add((steps - 1) % 2)
                wait_add((steps - 2) % 2)
            else:
                wait_add(0)
                wait_add(1)
            plsc.subcore_barrier()
            copy_out(p)
            plsc.subcore_barrier()

    return k(rows, ex2d, idx2d, zeros)


def _tconv_pass(q_tbl_src, q_ci, kv_src, kv_ci, elf, esm, a64, a5, didx3, nd):
    scale = 1.0 / math.sqrt(128.0)
    U, ex = _combine(q_tbl_src, q_ci, kv_src, kv_ci, elf, esm, a64, a5, scale)
    return _scatter_partials(U, ex, didx3, nd)


def _pad_out(w, b):
    return jnp.pad(w, ((0, 0), (0, 64))), jnp.pad(b, (0, 64))


def kernel(solvers, var_lp_f, con_lp_f, lo_costs, hi_costs, def_mm,
           edge_lp_f_wo_ss, var_learned_f, con_learned_f, edge_learned_f,
           edge_index_var_con, params, num_dual_iterations):
    vi = edge_index_var_con[0].reshape(_SC_W, -1, _CH)
    ci = edge_index_var_con[1].reshape(_SC_W, -1, _CH)

    vlf = var_learned_f
    clf = con_learned_f
    elf = edge_learned_f
    esm = jnp.concatenate(
        [lo_costs[:, None], hi_costs[:, None], edge_lp_f_wo_ss], axis=1)

    pred = params["pred"]
    n_layers = len(params["layers"])
    for li, lp in enumerate(params["layers"]):
        last = li == n_layers - 1
        vc = jnp.concatenate([vlf, var_lp_f], axis=1)
        cc = jnp.concatenate([clf, con_lp_f], axis=1)
        con, var, edge = lp["con"], lp["var"], lp["edge"]

        w_kvq_v = jnp.concatenate(
            [con["k"]["w"], con["v"]["w"], var["q"]["w"]], axis=1)
        b_kvq_v = jnp.concatenate(
            [con["k"]["b"], con["v"]["b"], var["q"]["b"]])
        tbl_v = _mm(vc, w_kvq_v, b_kvq_v)
        tbl_qc = _mm(cc, con["q"]["w"], con["q"]["b"])

        g_v = _gather_rows(tbl_v, vi)
        g_qc = _gather_rows(tbl_qc, ci)

        we = con["e"]["w"]
        pc = _tconv_pass(g_qc, 0, g_v, 0, elf, esm, we[:64], we[64:69], ci,
                         cc.shape[0])
        clf = _node_epilogue(pc, cc, con["skip"]["w"],
                             con["skip"]["b"], lp["cn"]["w"], lp["cn"]["b"])
        cc = jnp.concatenate([clf, con_lp_f], axis=1)

        w_kv_c = jnp.concatenate([var["k"]["w"], var["v"]["w"]], axis=1)
        b_kv_c = jnp.concatenate([var["k"]["b"], var["v"]["b"]])
        tbl_kvc = _mm(cc, w_kv_c, b_kv_c)
        if last:
            hc = _mlp2(cc, edge["c1"]["w"], edge["c1"]["b"],
                       edge["c2"]["w"], edge["c2"]["b"])
            hcp = _mlp2(cc, pred["c1"]["w"], pred["c1"]["b"],
                        pred["c2"]["w"], pred["c2"]["b"])
            tbl_c = jnp.concatenate([tbl_kvc, hc, hcp], axis=1)
        else:
            c2w, c2b = _pad_out(edge["c2"]["w"], edge["c2"]["b"])
            hc = _mlp2(cc, edge["c1"]["w"], edge["c1"]["b"], c2w, c2b)
            tbl_c = jnp.concatenate([tbl_kvc, hc], axis=1)
        g_c = _gather_rows(tbl_c, ci)

        we = var["e"]["w"]
        pv = _tconv_pass(g_v, 2, g_c, 0, elf, esm, we[:64], we[64:69], vi,
                         vc.shape[0])
        vlf = _node_epilogue(pv, vc, var["skip"]["w"],
                             var["skip"]["b"], lp["vn"]["w"], lp["vn"]["b"])
        vc = jnp.concatenate([vlf, var_lp_f], axis=1)

        if last:
            hva = _mlp2(vc, edge["v1"]["w"], edge["v1"]["b"],
                        edge["v2"]["w"], edge["v2"]["b"])
            hvp = _mlp2(vc, pred["v1"]["w"], pred["v1"]["b"],
                        pred["v2"]["w"], pred["v2"]["b"])
            hv = jnp.concatenate([hva, hvp], axis=1)
        else:
            v2w, v2b = _pad_out(edge["v2"]["w"], edge["v2"]["b"])
            hv = _mlp2(vc, edge["v1"]["w"], edge["v1"]["b"], v2w, v2b)
        g_hv = _gather_rows(hv, vi)
        e1 = edge["e1"]["w"]
        y, part = _edge_mlp(elf, esm, g_hv, 0, 128, 0, g_c, 2, 128, 0,
                            e1[:64], e1[64:69], e1[69:133], e1[133:197],
                            edge["e1"]["b"], edge["e2"]["w"], edge["e2"]["b"],
                            want_stats=True)
        elf = _ln_apply(y, part, lp["en"]["w"], lp["en"]["b"], _BE)

    e1 = pred["e1"]["w"]
    y, _ = _edge_mlp(elf, esm, g_hv, 0, 128, 64, g_c, 2, 128, 64,
                     e1[:64], e1[64:69], e1[69:133], e1[133:197],
                     pred["e1"]["b"], pred["e2"]["w"], pred["e2"]["b"],
                     want_stats=False)
    return y


# speedup vs baseline: 1.0765x; 1.0090x over previous
import functools
import math

import jax
import jax.numpy as jnp
from jax import lax
from jax.experimental import pallas as pl
from jax.experimental.pallas import tpu as pltpu

try:
    from jax.experimental.pallas import tpu_sc as plsc
except ImportError:
    plsc = None

F32 = jnp.float32


def _mm(x, w, b, act=False, bm=1000):
    M, K = x.shape
    N = w.shape[1]

    def kern(x_ref, w_ref, b_ref, o_ref):
        y = jnp.dot(x_ref[...], w_ref[...], preferred_element_type=F32)
        y = y + b_ref[...]
        if act:
            y = jnp.maximum(y, 0.0)
        o_ref[...] = y

    return pl.pallas_call(
        kern,
        grid=(M // bm,),
        in_specs=[
            pl.BlockSpec((bm, K), lambda i: (i, 0)),
            pl.BlockSpec((K, N), lambda i: (0, 0)),
            pl.BlockSpec((1, N), lambda i: (0, 0)),
        ],
        out_specs=pl.BlockSpec((bm, N), lambda i: (i, 0)),
        out_shape=jax.ShapeDtypeStruct((M, N), F32),
    )(x, w, b.reshape(1, -1))


def _mlp2(x, w1, b1, w2, b2, bm=1000):
    M, K = x.shape
    H = w1.shape[1]
    N = w2.shape[1]

    def kern(x_ref, w1_ref, b1_ref, w2_ref, b2_ref, o_ref):
        h = jnp.dot(x_ref[...], w1_ref[...], preferred_element_type=F32)
        h = jnp.maximum(h + b1_ref[...], 0.0)
        y = jnp.dot(h, w2_ref[...], preferred_element_type=F32)
        o_ref[...] = jnp.maximum(y + b2_ref[...], 0.0)

    return pl.pallas_call(
        kern,
        grid=(M // bm,),
        in_specs=[
            pl.BlockSpec((bm, K), lambda i: (i, 0)),
            pl.BlockSpec((K, H), lambda i: (0, 0)),
            pl.BlockSpec((1, H), lambda i: (0, 0)),
            pl.BlockSpec((H, N), lambda i: (0, 0)),
            pl.BlockSpec((1, N), lambda i: (0, 0)),
        ],
        out_specs=pl.BlockSpec((bm, N), lambda i: (i, 0)),
        out_shape=jax.ShapeDtypeStruct((M, N), F32),
    )(x, w1, b1.reshape(1, -1), w2, b2.reshape(1, -1))


_BE = 8000


def _combine(qsrc, q_ci, kvsrc, kv_ci, elf, esm, a64, a5, scale):
    E = elf.shape[0]
    C = 128

    def kern(q_ref, kv_ref, f_ref, s_ref, a64_ref, a5_ref, o_ref, x_ref):
        e = jnp.dot(f_ref[...], a64_ref[...], preferred_element_type=F32)
        e = e + jnp.dot(s_ref[...], a5_ref[...], preferred_element_type=F32)
        ke = kv_ref[:, :C]
        ve = kv_ref[:, C:]
        a = jnp.sum(q_ref[...] * (ke + e), axis=1) * scale
        ex = jnp.exp(a)
        o_ref[...] = (ve + e) * ex[:, None]
        li = lax.broadcasted_iota(jnp.int32, (_BE, 128), 1)
        x_ref[...] = jnp.where(li == 0, ex[:, None], 0.0) + jnp.where(
            li == 1, 1.0, 0.0)

    return pl.pallas_call(
        kern,
        grid=(E // _BE,),
        in_specs=[
            pl.BlockSpec((_BE, 128), lambda i, c=q_ci: (i, c)),
            pl.BlockSpec((_BE, 256), lambda i, c=kv_ci: (i, c)),
            pl.BlockSpec((_BE, 64), lambda i: (i, 0)),
            pl.BlockSpec((_BE, 5), lambda i: (i, 0)),
            pl.BlockSpec((64, 128), lambda i: (0, 0)),
            pl.BlockSpec((5, 128), lambda i: (0, 0)),
        ],
        out_specs=[
            pl.BlockSpec((_BE, 128), lambda i: (i, 0)),
            pl.BlockSpec((_BE, 128), lambda i: (i, 0)),
        ],
        out_shape=[
            jax.ShapeDtypeStruct((E, 128), F32),
            jax.ShapeDtypeStruct((E, 128), F32),
        ],
    )(qsrc, kvsrc, elf, esm, a64, a5)


def _node_epilogue(parts, xd, wskip, bskip, lnw, lnb, bm=1000):
    N = xd.shape[0]
    K = xd.shape[1]
    T = N // bm

    def kern_a(u0_ref, u1_ref, t0_ref, t1_ref, xd_ref, w_ref, b_ref,
               y_ref, pt_ref):
        U = u0_ref[0, 0] + u1_ref[0, 0]
        tail = t0_ref[0, 0] + t1_ref[0, 0]
        S = tail[:, 0:1]
        deg = tail[:, 1:2]
        agg = U / (S + 1e-16) / jnp.maximum(deg, 1.0)
        y = agg + jnp.dot(xd_ref[...], w_ref[...], preferred_element_type=F32) + b_ref[...]
        y_ref[...] = y
        sy = jnp.sum(y)
        sq = jnp.sum(y * y)
        li = lax.broadcasted_iota(jnp.int32, (1, 1, 128), 2)
        pt_ref[...] = jnp.where(li == 0, sy, 0.0) + jnp.where(li == 1, sq, 0.0)

    y, part = pl.pallas_call(
        kern_a,
        grid=(T,),
        in_specs=[
            pl.BlockSpec((1, 1, bm, 128), lambda i: (0, 0, i, 0)),
            pl.BlockSpec((1, 1, bm, 128), lambda i: (1, 0, i, 0)),
            pl.BlockSpec((1, 1, bm, 128), lambda i: (0, 1, i, 0)),
            pl.BlockSpec((1, 1, bm, 128), lambda i: (1, 1, i, 0)),
            pl.BlockSpec((bm, K), lambda i: (i, 0)),
            pl.BlockSpec((K, 128), lambda i: (0, 0)),
            pl.BlockSpec((1, 128), lambda i: (0, 0)),
        ],
        out_specs=[
            pl.BlockSpec((bm, 128), lambda i: (i, 0)),
            pl.BlockSpec((1, 1, 128), lambda i: (i, 0, 0)),
        ],
        out_shape=[
            jax.ShapeDtypeStruct((N, 128), F32),
            jax.ShapeDtypeStruct((T, 1, 128), F32),
        ],
    )(parts, parts, parts, parts, xd, wskip, bskip.reshape(1, -1))
    return _ln_apply(y, part, lnw, lnb, bm)


def _ln_apply(y, part, lnw, lnb, bm):
    N, D = y.shape
    T = part.shape[0]
    cnt = float(N * D)

    def kern(y_ref, pt_ref, w_ref, b_ref, o_ref):
        sy = jnp.sum(pt_ref[:, :, 0:1])
        sq = jnp.sum(pt_ref[:, :, 1:2])
        m = sy / cnt
        var = jnp.maximum(sq / cnt - m * m, 0.0)
        inv = 1.0 / (jnp.sqrt(var) + 1e-5)
        o_ref[...] = jnp.maximum((y_ref[...] - m) * inv * w_ref[..., :D] + b_ref[..., :D], 0.0)

    return pl.pallas_call(
        kern,
        grid=(N // bm,),
        in_specs=[
            pl.BlockSpec((bm, D), lambda i: (i, 0)),
            pl.BlockSpec((T, 1, 128), lambda i: (0, 0, 0)),
            pl.BlockSpec((1, 128), lambda i: (0, 0)),
            pl.BlockSpec((1, 128), lambda i: (0, 0)),
        ],
        out_specs=pl.BlockSpec((bm, D), lambda i: (i, 0)),
        out_shape=jax.ShapeDtypeStruct((N, D), F32),
    )(y, part, _pad128(lnw).reshape(1, -1), _pad128(lnb).reshape(1, -1))


def _pad128(v):
    d = v.shape[0]
    if d >= 128:
        return v
    return jnp.pad(v, (0, 128 - d))


def _edge_mlp(elf, esm, hvsrc, hv_ci, hv_w, hv_off, hcsrc, hc_ci, hc_w,
              hc_off, wf, w5, wv, wc, b1, w2, b2, want_stats):
    E = elf.shape[0]
    OE = w2.shape[1]
    T = E // _BE

    def kern(f_ref, s_ref, hv_ref, hc_ref, wf_ref, w5_ref, wv_ref, wc_ref,
             b1_ref, w2_ref, b2_ref, y_ref, pt_ref):
        h = jnp.dot(f_ref[...], wf_ref[...], preferred_element_type=F32)
        h = h + jnp.dot(s_ref[...], w5_ref[...], preferred_element_type=F32)
        h = h + jnp.dot(hv_ref[:, hv_off:hv_off + 64], wv_ref[...],
                        preferred_element_type=F32)
        h = h + jnp.dot(hc_ref[:, hc_off:hc_off + 64], wc_ref[...],
                        preferred_element_type=F32)
        h = jnp.maximum(h + b1_ref[...], 0.0)
        y = jnp.dot(h, w2_ref[...], preferred_element_type=F32) + b2_ref[...]
        y_ref[...] = y
        if want_stats:
            sy = jnp.sum(y)
            sq = jnp.sum(y * y)
            li = lax.broadcasted_iota(jnp.int32, (1, 1, 128), 2)
            pt_ref[...] = jnp.where(li == 0, sy, 0.0) + jnp.where(li == 1, sq, 0.0)
        else:
            pt_ref[...] = jnp.zeros((1, 1, 128), F32)

    return pl.pallas_call(
        kern,
        grid=(T,),
        in_specs=[
            pl.BlockSpec((_BE, 64), lambda i: (i, 0)),
            pl.BlockSpec((_BE, 5), lambda i: (i, 0)),
            pl.BlockSpec((_BE, hv_w), lambda i, c=hv_ci: (i, c)),
            pl.BlockSpec((_BE, hc_w), lambda i, c=hc_ci: (i, c)),
            pl.BlockSpec((64, 64), lambda i: (0, 0)),
            pl.BlockSpec((5, 64), lambda i: (0, 0)),
            pl.BlockSpec((64, 64), lambda i: (0, 0)),
            pl.BlockSpec((64, 64), lambda i: (0, 0)),
            pl.BlockSpec((1, 64), lambda i: (0, 0)),
            pl.BlockSpec((64, OE), lambda i: (0, 0)),
            pl.BlockSpec((1, OE), lambda i: (0, 0)),
        ],
        out_specs=[
            pl.BlockSpec((_BE, OE), lambda i: (i, 0)),
            pl.BlockSpec((1, 1, 128), lambda i: (i, 0, 0)),
        ],
        out_shape=[
            jax.ShapeDtypeStruct((E, OE), F32),
            jax.ShapeDtypeStruct((T, 1, 128), F32),
        ],
    )(elf, esm, hvsrc, hcsrc, wf, w5, wv, wc, b1.reshape(1, -1), w2,
      b2.reshape(1, -1))


_SC_CORES = 2
_SC_SUBCORES = 16
_SC_W = _SC_CORES * _SC_SUBCORES
_CH = 80


def _gather_rows(table, idx2d):
    N, D = table.shape
    W, steps, _ = idx2d.shape
    E = W * steps * _CH
    per_w = E // _SC_W
    mesh = plsc.VectorSubcoreMesh(core_axis_name="c", subcore_axis_name="s")

    @functools.partial(
        pl.kernel,
        mesh=mesh,
        out_type=jax.ShapeDtypeStruct((E, D), F32),
        scratch_types=[
            pltpu.VMEM((steps, _CH), jnp.int32),
            pltpu.VMEM((_CH, D), F32),
            pltpu.VMEM((_CH, D), F32),
            pltpu.SemaphoreType.DMA,
            pltpu.SemaphoreType.DMA,
        ],
    )
    def k(table_hbm, idx_hbm, out_hbm, idx_v, rows0, rows1, sem0, sem1):
        wid = lax.axis_index("s") * _SC_CORES + lax.axis_index("c")
        base = wid * per_w
        pltpu.sync_copy(idx_hbm.at[wid], idx_v)
        bufs = (rows0, rows1)
        sems = (sem0, sem1)

        def fire(j, b):
            pltpu.async_copy(table_hbm.at[idx_v.at[j]], bufs[b], sems[b])

        fire(0, 0)
        fire(1, 1)

        def body(g, carry):
            for b in range(2):
                j = 2 * g + b
                pltpu.make_async_copy(
                    table_hbm.at[pl.ds(0, _CH)], bufs[b], sems[b]).wait()
                pltpu.sync_copy(bufs[b], out_hbm.at[pl.ds(base + j * _CH, _CH)])

                @pl.when(j + 2 < steps)
                def _():
                    fire(j + 2, b)
            return carry

        lax.fori_loop(0, steps // 2, body, 0)
        if steps % 2:
            j = steps - 1
            pltpu.make_async_copy(
                table_hbm.at[pl.ds(0, _CH)], bufs[j % 2], sems[j % 2]).wait()
            pltpu.sync_copy(bufs[j % 2], out_hbm.at[pl.ds(base + j * _CH, _CH)])

    return k(table, idx2d)


def _scatter_partials(rows, ex2d, idx2d, N):
    E, D = rows.shape
    per_w = E // _SC_W
    steps = idx2d.shape[1]
    cp = (N // _SC_SUBCORES) & ~7
    rem = N - cp * _SC_SUBCORES
    zeros = jnp.zeros((N, 128), F32)
    mesh = plsc.VectorSubcoreMesh(core_axis_name="c", subcore_axis_name="s")

    @functools.partial(
        pl.kernel,
        mesh=mesh,
        out_type=jax.ShapeDtypeStruct((_SC_CORES, 2, N, 128), F32),
        scratch_types=[
            pltpu.VMEM((steps, _CH), jnp.int32),
            pltpu.VMEM((_CH, 128), F32),
            pltpu.VMEM((_CH, 128), F32),
            pltpu.VMEM_SHARED((N, 128), F32),
            pltpu.SemaphoreType.DMA,
            pltpu.SemaphoreType.DMA,
            pltpu.SemaphoreType.DMA,
            pltpu.SemaphoreType.DMA,
        ],
    )
    def k(rows_hbm, ex_hbm, idx_hbm, zero_hbm, out_hbm,
          idx_v, rows0, rows1, acc_sh, sem0, sem1, asem0, asem1):
        cid = lax.axis_index("c")
        sid = lax.axis_index("s")
        wid = sid * _SC_CORES + cid
        base = wid * per_w
        pltpu.sync_copy(idx_hbm.at[wid], idx_v)
        bufs = (rows0, rows1)
        sems = (sem0, sem1)
        asems = (asem0, asem1)

        def copy_out(p):
            dst = out_hbm.at[cid].at[p]
            pltpu.sync_copy(
                acc_sh.at[pl.ds(sid * cp, cp)],
                dst.at[pl.ds(sid * cp, cp)],
            )
            if rem:
                @pl.when(sid == 0)
                def _():
                    pltpu.sync_copy(
                        acc_sh.at[pl.ds(cp * _SC_SUBCORES, rem)],
                        dst.at[pl.ds(cp * _SC_SUBCORES, rem)],
                    )

        for p, src in enumerate((rows_hbm, ex_hbm)):
            @pl.when(sid == 0)
            def _():
                pltpu.sync_copy(zero_hbm, acc_sh)

            plsc.subcore_barrier()

            def fire(j, b, src=src):
                pltpu.async_copy(
                    src.at[pl.ds(base + j * _CH, _CH)], bufs[b], sems[b])

            def drain_add(j, b, src=src):
                pltpu.make_async_copy(
                    src.at[pl.ds(0, _CH)], bufs[b], sems[b]).wait()
                pltpu.async_copy(
                    bufs[b], acc_sh.at[idx_v.at[j]], asems[b], add=True)

            def wait_add(b, src=src):
                pltpu.make_async_copy(
                    src.at[pl.ds(0, _CH)], bufs[b], asems[b]).wait()

            fire(0, 0)
            fire(1, 1)

            def body(g, carry, fire=fire, drain_add=drain_add,
                     wait_add=wait_add):
                for b in range(2):
                    j = 2 * g + b
                    drain_add(j, b)

                    @pl.when(j + 2 < steps)
                    def _():
                        wait_add(b)
                        fire(j + 2, b)
                return carry

            lax.fori_loop(0, steps // 2, body, 0)
            if steps % 2:
                drain_add(steps - 1, (steps - 1) % 2)
                wait_add((steps - 1) % 2)
                wait_add((steps - 2) % 2)
            else:
                wait_add(0)
                wait_add(1)
            plsc.subcore_barrier()
            copy_out(p)
            plsc.subcore_barrier()

    return k(rows, ex2d, idx2d, zeros)


def _tconv_pass(q_tbl_src, q_ci, kv_src, kv_ci, elf, esm, a64, a5, didx3, nd):
    scale = 1.0 / math.sqrt(128.0)
    U, ex = _combine(q_tbl_src, q_ci, kv_src, kv_ci, elf, esm, a64, a5, scale)
    return _scatter_partials(U, ex, didx3, nd)


def _pad_out(w, b):
    return jnp.pad(w, ((0, 0), (0, 64))), jnp.pad(b, (0, 64))


def kernel(solvers, var_lp_f, con_lp_f, lo_costs, hi_costs, def_mm,
           edge_lp_f_wo_ss, var_learned_f, con_learned_f, edge_learned_f,
           edge_index_var_con, params, num_dual_iterations):
    vi = edge_index_var_con[0].reshape(_SC_W, -1, _CH)
    ci = edge_index_var_con[1].reshape(_SC_W, -1, _CH)

    vlf = var_learned_f
    clf = con_learned_f
    elf = edge_learned_f
    esm = jnp.concatenate(
        [lo_costs[:, None], hi_costs[:, None], edge_lp_f_wo_ss], axis=1)

    pred = params["pred"]
    n_layers = len(params["layers"])
    for li, lp in enumerate(params["layers"]):
        last = li == n_layers - 1
        vc = jnp.concatenate([vlf, var_lp_f], axis=1)
        cc = jnp.concatenate([clf, con_lp_f], axis=1)
        con, var, edge = lp["con"], lp["var"], lp["edge"]

        w_kvq_v = jnp.concatenate(
            [con["k"]["w"], con["v"]["w"], var["q"]["w"]], axis=1)
        b_kvq_v = jnp.concatenate(
            [con["k"]["b"], con["v"]["b"], var["q"]["b"]])
        tbl_v = _mm(vc, w_kvq_v, b_kvq_v)
        tbl_qc = _mm(cc, con["q"]["w"], con["q"]["b"])

        g_v = _gather_rows(tbl_v, vi)
        g_qc = _gather_rows(tbl_qc, ci)

        we = con["e"]["w"]
        pc = _tconv_pass(g_qc, 0, g_v, 0, elf, esm, we[:64], we[64:69], ci,
                         cc.shape[0])
        clf = _node_epilogue(pc, cc, con["skip"]["w"],
                             con["skip"]["b"], lp["cn"]["w"], lp["cn"]["b"])
        cc = jnp.concatenate([clf, con_lp_f], axis=1)

        w_kv_c = jnp.concatenate([var["k"]["w"], var["v"]["w"]], axis=1)
        b_kv_c = jnp.concatenate([var["k"]["b"], var["v"]["b"]])
        tbl_kvc = _mm(cc, w_kv_c, b_kv_c)
        if last:
            hc = _mlp2(cc, edge["c1"]["w"], edge["c1"]["b"],
                       edge["c2"]["w"], edge["c2"]["b"])
            hcp = _mlp2(cc, pred["c1"]["w"], pred["c1"]["b"],
                        pred["c2"]["w"], pred["c2"]["b"])
            tbl_c = jnp.concatenate([tbl_kvc, hc, hcp], axis=1)
        else:
            c2w, c2b = _pad_out(edge["c2"]["w"], edge["c2"]["b"])
            hc = _mlp2(cc, edge["c1"]["w"], edge["c1"]["b"], c2w, c2b)
            tbl_c = jnp.concatenate([tbl_kvc, hc], axis=1)
        g_c = _gather_rows(tbl_c, ci)

        we = var["e"]["w"]
        pv = _tconv_pass(g_v, 2, g_c, 0, elf, esm, we[:64], we[64:69], vi,
                         vc.shape[0])
        vlf = _node_epilogue(pv, vc, var["skip"]["w"],
                             var["skip"]["b"], lp["vn"]["w"], lp["vn"]["b"])
        vc = jnp.concatenate([vlf, var_lp_f], axis=1)

        if last:
            hva = _mlp2(vc, edge["v1"]["w"], edge["v1"]["b"],
                        edge["v2"]["w"], edge["v2"]["b"])
            hvp = _mlp2(vc, pred["v1"]["w"], pred["v1"]["b"],
                        pred["v2"]["w"], pred["v2"]["b"])
            hv = jnp.concatenate([hva, hvp], axis=1)
        else:
            v2w, v2b = _pad_out(edge["v2"]["w"], edge["v2"]["b"])
            hv = _mlp2(vc, edge["v1"]["w"], edge["v1"]["b"], v2w, v2b)
        g_hv = _gather_rows(hv, vi)
        e1 = edge["e1"]["w"]
        y, part = _edge_mlp(elf, esm, g_hv, 0, 128, 0, g_c, 2, 128, 0,
                            e1[:64], e1[64:69], e1[69:133], e1[133:197],
                            edge["e1"]["b"], edge["e2"]["w"], edge["e2"]["b"],
                            want_stats=True)
        elf = _ln_apply(y, part, lp["en"]["w"], lp["en"]["b"], _BE)

    e1 = pred["e1"]["w"]
    y, _ = _edge_mlp(elf, esm, g_hv, 0, 128, 64, g_c, 2, 128, 64,
                     e1[:64], e1[64:69], e1[69:133], e1[133:197],
                     pred["e1"]["b"], pred["e2"]["w"], pred["e2"]["b"],
                     want_stats=False)
    return y
